# baseline (speedup 1.0000x reference)
import functools
import jax
import jax.numpy as jnp
from jax import lax
from jax.experimental import pallas as pl
from jax.experimental.pallas import tpu as pltpu
from jax.experimental.pallas import tpu_sc as plsc

VOCAB_N = 1_000_000
BATCH_N = 16384
HIST_N = 200
TOTAL_ITEMS = BATCH_N * HIST_N

NUM_CORES = 2
NUM_SUBCORES = 16
NW = NUM_CORES * NUM_SUBCORES

ITEMS_PER_W = TOTAL_ITEMS // NW
CHUNK = 12_800
N_CHUNKS = ITEMS_PER_W // CHUNK
SUB = 1600
N_SUB = CHUNK // SUB

CLIENTS_PER_W = BATCH_N // NW

STAGE_PER_SUB = 62_496
STAGE_CHUNK = 15_624
N_STAGE = STAGE_PER_SUB // STAGE_CHUNK
STAGE_TAIL = VOCAB_N - NUM_SUBCORES * STAGE_PER_SUB

_mesh = plsc.VectorSubcoreMesh(core_axis_name="c", subcore_axis_name="s")


@functools.partial(
    pl.kernel,
    out_type=(
        jax.ShapeDtypeStruct((BATCH_N,), jnp.float32),
        jax.ShapeDtypeStruct((TOTAL_ITEMS,), jnp.float32),
    ),
    mesh=_mesh,
    scratch_types=[
        pltpu.VMEM_SHARED((VOCAB_N,), jnp.float32),
        pltpu.VMEM((CHUNK,), jnp.int32),
        pltpu.VMEM((CHUNK,), jnp.int32),
        pltpu.VMEM((CHUNK,), jnp.float32),
        pltpu.VMEM((CHUNK,), jnp.float32),
        pltpu.VMEM((CLIENTS_PER_W,), jnp.int32),
        pltpu.VMEM((CLIENTS_PER_W,), jnp.float32),
        pltpu.VMEM((STAGE_CHUNK,), jnp.float32),
        pltpu.SemaphoreType.DMA,
        pltpu.SemaphoreType.DMA,
        pltpu.SemaphoreType.DMA,
        pltpu.SemaphoreType.DMA,
        pltpu.SemaphoreType.DMA,
    ],
)
def _gather_kernel(
    item_tab_hbm,
    client_tab_hbm,
    client_ids_hbm,
    item_ids_hbm,
    out_client_hbm,
    out_items_hbm,
    item_sp,
    idx0_v,
    idx1_v,
    val0_v,
    val1_v,
    cidx_v,
    cval_v,
    stage_v,
    si0,
    si1,
    so0,
    so1,
    sg1,
):
    c = lax.axis_index("c")
    s = lax.axis_index("s")
    wid = s * NUM_CORES + c
    base = wid * ITEMS_PER_W

    pltpu.async_copy(item_ids_hbm.at[pl.ds(base, CHUNK)], idx0_v, si0)
    pltpu.async_copy(item_ids_hbm.at[pl.ds(base + CHUNK, CHUNK)], idx1_v, si1)

    base_off = s * STAGE_PER_SUB

    @pl.loop(0, N_STAGE)
    def _st(j):
        off = base_off + j * STAGE_CHUNK
        pltpu.sync_copy(item_tab_hbm.at[pl.ds(off, STAGE_CHUNK)], stage_v)
        pltpu.sync_copy(stage_v, item_sp.at[pl.ds(off, STAGE_CHUNK)])

    @pl.when(s == 0)
    def _st_tail():
        toff = NUM_SUBCORES * STAGE_PER_SUB
        pltpu.sync_copy(
            item_tab_hbm.at[pl.ds(toff, STAGE_TAIL)],
            stage_v.at[pl.ds(0, STAGE_TAIL)],
        )
        pltpu.sync_copy(
            stage_v.at[pl.ds(0, STAGE_TAIL)],
            item_sp.at[pl.ds(toff, STAGE_TAIL)],
        )

    cbase = wid * CLIENTS_PER_W
    pltpu.sync_copy(client_ids_hbm.at[pl.ds(cbase, CLIENTS_PER_W)], cidx_v)
    pltpu.sync_copy(client_tab_hbm.at[cidx_v], cval_v)
    pltpu.sync_copy(cval_v, out_client_hbm.at[pl.ds(cbase, CLIENTS_PER_W)])

    plsc.subcore_barrier()

    @pl.loop(0, N_CHUNKS, step=2)
    def _chunk(i):
        off0 = base + i * CHUNK
        off1 = off0 + CHUNK

        pltpu.make_async_copy(
            item_ids_hbm.at[pl.ds(off0, CHUNK)], idx0_v, si0
        ).wait()
        pltpu.make_async_copy(
            item_ids_hbm.at[pl.ds(off1, CHUNK)], idx1_v, si1
        ).wait()

        @pl.when(i > 0)
        def _wait_out():
            pltpu.make_async_copy(
                val0_v, out_items_hbm.at[pl.ds(off0 - 2 * CHUNK, CHUNK)], so0
            ).wait()
            pltpu.make_async_copy(
                val1_v, out_items_hbm.at[pl.ds(off1 - 2 * CHUNK, CHUNK)], so1
            ).wait()

        for j in range(N_SUB):
            sl = pl.ds(j * SUB, SUB)
            pltpu.async_copy(
                item_tab_hbm.at[idx1_v.at[sl]], val1_v.at[sl], sg1
            )
        pltpu.sync_copy(item_sp.at[idx0_v], val0_v)
        for j in range(N_SUB):
            sl = pl.ds(j * SUB, SUB)
            pltpu.make_async_copy(
                item_tab_hbm.at[idx1_v.at[sl]], val1_v.at[sl], sg1
            ).wait()

        pltpu.async_copy(val0_v, out_items_hbm.at[pl.ds(off0, CHUNK)], so0)
        pltpu.async_copy(val1_v, out_items_hbm.at[pl.ds(off1, CHUNK)], so1)

        @pl.when(i + 2 < N_CHUNKS)
        def _prefetch():
            pltpu.async_copy(
                item_ids_hbm.at[pl.ds(off0 + 2 * CHUNK, CHUNK)], idx0_v, si0
            )
            pltpu.async_copy(
                item_ids_hbm.at[pl.ds(off1 + 2 * CHUNK, CHUNK)], idx1_v, si1
            )

    pltpu.make_async_copy(
        val0_v, out_items_hbm.at[pl.ds(base + (N_CHUNKS - 2) * CHUNK, CHUNK)], so0
    ).wait()
    pltpu.make_async_copy(
        val1_v, out_items_hbm.at[pl.ds(base + (N_CHUNKS - 1) * CHUNK, CHUNK)], so1
    ).wait()


def kernel(item_id2graph_id, client_id2graph_id, client_ids, item_ids):
    flat_items = item_ids.reshape(-1)
    out_client, out_items = _gather_kernel(
        item_id2graph_id, client_id2graph_id, client_ids, flat_items
    )
    return (out_client, out_items.reshape(BATCH_N, HIST_N))


# speedup vs baseline: 1.1530x; 1.1530x over previous
import functools
import jax
import jax.numpy as jnp
from jax import lax
from jax.experimental import pallas as pl
from jax.experimental.pallas import tpu as pltpu
from jax.experimental.pallas import tpu_sc as plsc

VOCAB_N = 1_000_000
BATCH_N = 16384
HIST_N = 200
TOTAL_ITEMS = BATCH_N * HIST_N

NUM_CORES = 2
NUM_SUBCORES = 16
NW = NUM_CORES * NUM_SUBCORES

ITEMS_PER_W = TOTAL_ITEMS // NW
N_IT = 8
PAIR = ITEMS_PER_W // N_IT
S_CH = 9_344
H_CH = PAIR - S_CH
H_SUB = 1_728
N_HSUB = H_CH // H_SUB

CLIENTS_PER_W = BATCH_N // NW

STAGE_PER_SUB = 62_496
STAGE_CHUNK = 5_208
N_STAGE = STAGE_PER_SUB // STAGE_CHUNK
STAGE_TAIL = VOCAB_N - NUM_SUBCORES * STAGE_PER_SUB

_mesh = plsc.VectorSubcoreMesh(core_axis_name="c", subcore_axis_name="s")


@functools.partial(
    pl.kernel,
    out_type=(
        jax.ShapeDtypeStruct((BATCH_N,), jnp.float32),
        jax.ShapeDtypeStruct((TOTAL_ITEMS,), jnp.float32),
    ),
    mesh=_mesh,
    scratch_types=[
        pltpu.VMEM_SHARED((VOCAB_N,), jnp.float32),
        pltpu.VMEM((S_CH,), jnp.int32),
        pltpu.VMEM((S_CH,), jnp.int32),
        pltpu.VMEM((S_CH,), jnp.float32),
        pltpu.VMEM((S_CH,), jnp.float32),
        pltpu.VMEM((H_CH,), jnp.int32),
        pltpu.VMEM((H_CH,), jnp.int32),
        pltpu.VMEM((H_CH,), jnp.float32),
        pltpu.VMEM((H_CH,), jnp.float32),
        pltpu.VMEM((CLIENTS_PER_W,), jnp.int32),
        pltpu.VMEM((CLIENTS_PER_W,), jnp.float32),
        pltpu.VMEM((STAGE_CHUNK,), jnp.float32),
        pltpu.SemaphoreType.DMA,
        pltpu.SemaphoreType.DMA,
        pltpu.SemaphoreType.DMA,
        pltpu.SemaphoreType.DMA,
        pltpu.SemaphoreType.DMA,
        pltpu.SemaphoreType.DMA,
        pltpu.SemaphoreType.DMA,
        pltpu.SemaphoreType.DMA,
        pltpu.SemaphoreType.DMA,
        pltpu.SemaphoreType.DMA,
    ],
)
def _gather_kernel(
    item_tab_hbm,
    client_tab_hbm,
    client_ids_hbm,
    item_ids_hbm,
    out_client_hbm,
    out_items_hbm,
    item_sp,
    sidx0,
    sidx1,
    sval0,
    sval1,
    hidx0,
    hidx1,
    hval0,
    hval1,
    cidx_v,
    cval_v,
    stage_v,
    ssi0,
    ssi1,
    shi0,
    shi1,
    sso0,
    sso1,
    sho0,
    sho1,
    sg0,
    sg1,
):
    c = lax.axis_index("c")
    s = lax.axis_index("s")
    wid = s * NUM_CORES + c
    base = wid * ITEMS_PER_W

    def s_off(i):
        return base + i * PAIR

    def h_off(i):
        return base + i * PAIR + S_CH

    pltpu.async_copy(item_ids_hbm.at[pl.ds(s_off(0), S_CH)], sidx0, ssi0)
    pltpu.async_copy(item_ids_hbm.at[pl.ds(h_off(0), H_CH)], hidx0, shi0)
    pltpu.async_copy(item_ids_hbm.at[pl.ds(s_off(1), S_CH)], sidx1, ssi1)
    pltpu.async_copy(item_ids_hbm.at[pl.ds(h_off(1), H_CH)], hidx1, shi1)

    pltpu.make_async_copy(
        item_ids_hbm.at[pl.ds(h_off(0), H_CH)], hidx0, shi0
    ).wait()
    for j in range(N_HSUB):
        sl = pl.ds(j * H_SUB, H_SUB)
        pltpu.async_copy(item_tab_hbm.at[hidx0.at[sl]], hval0.at[sl], sg0)

    base_off = s * STAGE_PER_SUB

    @pl.loop(0, N_STAGE)
    def _st(j):
        off = base_off + j * STAGE_CHUNK
        pltpu.sync_copy(item_tab_hbm.at[pl.ds(off, STAGE_CHUNK)], stage_v)
        pltpu.sync_copy(stage_v, item_sp.at[pl.ds(off, STAGE_CHUNK)])

    @pl.when(s == 0)
    def _st_tail():
        toff = NUM_SUBCORES * STAGE_PER_SUB
        pltpu.sync_copy(
            item_tab_hbm.at[pl.ds(toff, STAGE_TAIL)],
            stage_v.at[pl.ds(0, STAGE_TAIL)],
        )
        pltpu.sync_copy(
            stage_v.at[pl.ds(0, STAGE_TAIL)],
            item_sp.at[pl.ds(toff, STAGE_TAIL)],
        )

    cbase = wid * CLIENTS_PER_W
    pltpu.sync_copy(client_ids_hbm.at[pl.ds(cbase, CLIENTS_PER_W)], cidx_v)
    pltpu.sync_copy(client_tab_hbm.at[cidx_v], cval_v)
    pltpu.sync_copy(cval_v, out_client_hbm.at[pl.ds(cbase, CLIENTS_PER_W)])

    plsc.subcore_barrier()

    def fire_hbm_dyn(hi_ref, hv_ref, sem):
        for j in range(N_HSUB):
            sl = pl.ds(j * H_SUB, H_SUB)
            pltpu.async_copy(item_tab_hbm.at[hi_ref.at[sl]], hv_ref.at[sl], sem)

    def drain_hbm_dyn(hi_ref, hv_ref, sem):
        for j in range(N_HSUB):
            sl = pl.ds(j * H_SUB, H_SUB)
            pltpu.make_async_copy(
                item_tab_hbm.at[hi_ref.at[sl]], hv_ref.at[sl], sem
            ).wait()

    @pl.loop(0, N_IT, step=2)
    def _pair(i):
        pltpu.make_async_copy(
            item_ids_hbm.at[pl.ds(s_off(i), S_CH)], sidx0, ssi0
        ).wait()

        @pl.when(i > 0)
        def _w_sval0():
            pltpu.make_async_copy(
                sval0, out_items_hbm.at[pl.ds(s_off(i - 2), S_CH)], sso0
            ).wait()

        pltpu.sync_copy(item_sp.at[sidx0], sval0)
        drain_hbm_dyn(hidx0, hval0, sg0)

        pltpu.async_copy(sval0, out_items_hbm.at[pl.ds(s_off(i), S_CH)], sso0)
        pltpu.async_copy(hval0, out_items_hbm.at[pl.ds(h_off(i), H_CH)], sho0)

        pltpu.make_async_copy(
            item_ids_hbm.at[pl.ds(h_off(i + 1), H_CH)], hidx1, shi1
        ).wait()

        @pl.when(i > 0)
        def _w_hval1():
            pltpu.make_async_copy(
                hval1, out_items_hbm.at[pl.ds(h_off(i - 1), H_CH)], sho1
            ).wait()

        fire_hbm_dyn(hidx1, hval1, sg1)

        @pl.when(i + 2 < N_IT)
        def _pf_a():
            pltpu.async_copy(
                item_ids_hbm.at[pl.ds(s_off(i + 2), S_CH)], sidx0, ssi0
            )
            pltpu.async_copy(
                item_ids_hbm.at[pl.ds(h_off(i + 2), H_CH)], hidx0, shi0
            )

        pltpu.make_async_copy(
            item_ids_hbm.at[pl.ds(s_off(i + 1), S_CH)], sidx1, ssi1
        ).wait()

        @pl.when(i > 0)
        def _w_sval1():
            pltpu.make_async_copy(
                sval1, out_items_hbm.at[pl.ds(s_off(i - 1), S_CH)], sso1
            ).wait()

        pltpu.sync_copy(item_sp.at[sidx1], sval1)
        drain_hbm_dyn(hidx1, hval1, sg1)

        pltpu.async_copy(
            sval1, out_items_hbm.at[pl.ds(s_off(i + 1), S_CH)], sso1
        )
        pltpu.async_copy(
            hval1, out_items_hbm.at[pl.ds(h_off(i + 1), H_CH)], sho1
        )

        @pl.when(i + 2 < N_IT)
        def _fire_next_a():
            pltpu.make_async_copy(
                item_ids_hbm.at[pl.ds(h_off(i + 2), H_CH)], hidx0, shi0
            ).wait()
            pltpu.make_async_copy(
                hval0, out_items_hbm.at[pl.ds(h_off(i), H_CH)], sho0
            ).wait()
            fire_hbm_dyn(hidx0, hval0, sg0)

            @pl.when(i + 3 < N_IT)
            def _pf_b():
                pltpu.async_copy(
                    item_ids_hbm.at[pl.ds(s_off(i + 3), S_CH)], sidx1, ssi1
                )
                pltpu.async_copy(
                    item_ids_hbm.at[pl.ds(h_off(i + 3), H_CH)], hidx1, shi1
                )

    pltpu.make_async_copy(
        sval0, out_items_hbm.at[pl.ds(s_off(N_IT - 2), S_CH)], sso0
    ).wait()
    pltpu.make_async_copy(
        hval0, out_items_hbm.at[pl.ds(h_off(N_IT - 2), H_CH)], sho0
    ).wait()
    pltpu.make_async_copy(
        sval1, out_items_hbm.at[pl.ds(s_off(N_IT - 1), S_CH)], sso1
    ).wait()
    pltpu.make_async_copy(
        hval1, out_items_hbm.at[pl.ds(h_off(N_IT - 1), H_CH)], sho1
    ).wait()


def kernel(item_id2graph_id, client_id2graph_id, client_ids, item_ids):
    flat_items = item_ids.reshape(-1)
    out_client, out_items = _gather_kernel(
        item_id2graph_id, client_id2graph_id, client_ids, flat_items
    )
    return (out_client, out_items.reshape(BATCH_N, HIST_N))


# speedup vs baseline: 1.1961x; 1.0374x over previous
import functools
import jax
import jax.numpy as jnp
from jax import lax
from jax.experimental import pallas as pl
from jax.experimental.pallas import tpu as pltpu
from jax.experimental.pallas import tpu_sc as plsc

VOCAB_N = 1_000_000
BATCH_N = 16384
HIST_N = 200
TOTAL_ITEMS = BATCH_N * HIST_N

NUM_CORES = 2
NUM_SUBCORES = 16
NW = NUM_CORES * NUM_SUBCORES

ITEMS_PER_W = TOTAL_ITEMS // NW
CHUNK = 12_800
N_CHUNKS = ITEMS_PER_W // CHUNK
H_SUB = 1_600
N_HSUB = CHUNK // H_SUB

CLIENTS_PER_W = BATCH_N // NW

STAGE_PER_SUB = 62_496
STAGE_CHUNK = 5_208
N_STAGE = STAGE_PER_SUB // STAGE_CHUNK
STAGE_TAIL = VOCAB_N - NUM_SUBCORES * STAGE_PER_SUB

_mesh = plsc.VectorSubcoreMesh(core_axis_name="c", subcore_axis_name="s")


@functools.partial(
    pl.kernel,
    out_type=(
        jax.ShapeDtypeStruct((BATCH_N,), jnp.float32),
        jax.ShapeDtypeStruct((TOTAL_ITEMS,), jnp.float32),
    ),
    mesh=_mesh,
    scratch_types=[
        pltpu.VMEM_SHARED((VOCAB_N,), jnp.float32),
        pltpu.VMEM((CHUNK,), jnp.int32),
        pltpu.VMEM((CHUNK,), jnp.int32),
        pltpu.VMEM((CHUNK,), jnp.float32),
        pltpu.VMEM((CHUNK,), jnp.float32),
        pltpu.VMEM((CLIENTS_PER_W,), jnp.int32),
        pltpu.VMEM((CLIENTS_PER_W,), jnp.float32),
        pltpu.VMEM((STAGE_CHUNK,), jnp.float32),
        pltpu.VMEM((STAGE_CHUNK,), jnp.float32),
        pltpu.SemaphoreType.DMA,
        pltpu.SemaphoreType.DMA,
        pltpu.SemaphoreType.DMA,
        pltpu.SemaphoreType.DMA,
        pltpu.SemaphoreType.DMA,
        pltpu.SemaphoreType.DMA,
        pltpu.SemaphoreType.DMA,
    ],
)
def _gather_kernel(
    item_tab_hbm,
    client_tab_hbm,
    client_ids_hbm,
    item_ids_hbm,
    out_client_hbm,
    out_items_hbm,
    item_sp,
    idx0_v,
    idx1_v,
    val0_v,
    val1_v,
    cidx_v,
    cval_v,
    stA_v,
    stB_v,
    si0,
    si1,
    so0,
    so1,
    sg0,
    stoA,
    stoB,
):
    c = lax.axis_index("c")
    s = lax.axis_index("s")
    wid = s * NUM_CORES + c
    base = wid * ITEMS_PER_W

    pltpu.async_copy(item_ids_hbm.at[pl.ds(base, CHUNK)], idx0_v, si0)
    pltpu.async_copy(item_ids_hbm.at[pl.ds(base + CHUNK, CHUNK)], idx1_v, si1)

    pltpu.make_async_copy(item_ids_hbm.at[pl.ds(base, CHUNK)], idx0_v, si0).wait()
    for j in range(N_HSUB):
        sl = pl.ds(j * H_SUB, H_SUB)
        pltpu.async_copy(item_tab_hbm.at[idx0_v.at[sl]], val0_v.at[sl], sg0)

    cbase = wid * CLIENTS_PER_W
    pltpu.sync_copy(client_ids_hbm.at[pl.ds(cbase, CLIENTS_PER_W)], cidx_v)
    pltpu.sync_copy(client_tab_hbm.at[cidx_v], cval_v)
    pltpu.sync_copy(cval_v, out_client_hbm.at[pl.ds(cbase, CLIENTS_PER_W)])

    base_off = s * STAGE_PER_SUB

    @pl.loop(0, N_STAGE, step=2)
    def _st(j):
        offA = base_off + j * STAGE_CHUNK
        offB = offA + STAGE_CHUNK

        @pl.when(j > 0)
        def _wA():
            pltpu.make_async_copy(
                stA_v, item_sp.at[pl.ds(offA - 2 * STAGE_CHUNK, STAGE_CHUNK)], stoA
            ).wait()

        pltpu.sync_copy(item_tab_hbm.at[pl.ds(offA, STAGE_CHUNK)], stA_v)
        pltpu.async_copy(stA_v, item_sp.at[pl.ds(offA, STAGE_CHUNK)], stoA)

        @pl.when(j > 0)
        def _wB():
            pltpu.make_async_copy(
                stB_v, item_sp.at[pl.ds(offB - 2 * STAGE_CHUNK, STAGE_CHUNK)], stoB
            ).wait()

        pltpu.sync_copy(item_tab_hbm.at[pl.ds(offB, STAGE_CHUNK)], stB_v)
        pltpu.async_copy(stB_v, item_sp.at[pl.ds(offB, STAGE_CHUNK)], stoB)

    pltpu.make_async_copy(
        stA_v,
        item_sp.at[pl.ds(base_off + (N_STAGE - 2) * STAGE_CHUNK, STAGE_CHUNK)],
        stoA,
    ).wait()
    pltpu.make_async_copy(
        stB_v,
        item_sp.at[pl.ds(base_off + (N_STAGE - 1) * STAGE_CHUNK, STAGE_CHUNK)],
        stoB,
    ).wait()

    @pl.when(s == 0)
    def _st_tail():
        toff = NUM_SUBCORES * STAGE_PER_SUB
        pltpu.sync_copy(
            item_tab_hbm.at[pl.ds(toff, STAGE_TAIL)],
            stA_v.at[pl.ds(0, STAGE_TAIL)],
        )
        pltpu.sync_copy(
            stA_v.at[pl.ds(0, STAGE_TAIL)],
            item_sp.at[pl.ds(toff, STAGE_TAIL)],
        )

    plsc.subcore_barrier()

    for j in range(N_HSUB):
        sl = pl.ds(j * H_SUB, H_SUB)
        pltpu.make_async_copy(
            item_tab_hbm.at[idx0_v.at[sl]], val0_v.at[sl], sg0
        ).wait()
    pltpu.async_copy(val0_v, out_items_hbm.at[pl.ds(base, CHUNK)], so0)
    pltpu.async_copy(item_ids_hbm.at[pl.ds(base + 2 * CHUNK, CHUNK)], idx0_v, si0)

    pltpu.make_async_copy(
        item_ids_hbm.at[pl.ds(base + CHUNK, CHUNK)], idx1_v, si1
    ).wait()
    pltpu.sync_copy(item_sp.at[idx1_v], val1_v)
    pltpu.async_copy(val1_v, out_items_hbm.at[pl.ds(base + CHUNK, CHUNK)], so1)
    pltpu.async_copy(item_ids_hbm.at[pl.ds(base + 3 * CHUNK, CHUNK)], idx1_v, si1)

    @pl.loop(2, N_CHUNKS, step=2)
    def _chunk(i):
        off0 = base + i * CHUNK
        off1 = off0 + CHUNK

        pltpu.make_async_copy(
            item_ids_hbm.at[pl.ds(off0, CHUNK)], idx0_v, si0
        ).wait()
        pltpu.make_async_copy(
            val0_v, out_items_hbm.at[pl.ds(off0 - 2 * CHUNK, CHUNK)], so0
        ).wait()
        pltpu.sync_copy(item_sp.at[idx0_v], val0_v)
        pltpu.async_copy(val0_v, out_items_hbm.at[pl.ds(off0, CHUNK)], so0)

        @pl.when(i + 2 < N_CHUNKS)
        def _pf0():
            pltpu.async_copy(
                item_ids_hbm.at[pl.ds(off0 + 2 * CHUNK, CHUNK)], idx0_v, si0
            )

        pltpu.make_async_copy(
            item_ids_hbm.at[pl.ds(off1, CHUNK)], idx1_v, si1
        ).wait()
        pltpu.make_async_copy(
            val1_v, out_items_hbm.at[pl.ds(off1 - 2 * CHUNK, CHUNK)], so1
        ).wait()
        pltpu.sync_copy(item_sp.at[idx1_v], val1_v)
        pltpu.async_copy(val1_v, out_items_hbm.at[pl.ds(off1, CHUNK)], so1)

        @pl.when(i + 2 < N_CHUNKS)
        def _pf1():
            pltpu.async_copy(
                item_ids_hbm.at[pl.ds(off1 + 2 * CHUNK, CHUNK)], idx1_v, si1
            )

    pltpu.make_async_copy(
        val0_v, out_items_hbm.at[pl.ds(base + (N_CHUNKS - 2) * CHUNK, CHUNK)], so0
    ).wait()
    pltpu.make_async_copy(
        val1_v, out_items_hbm.at[pl.ds(base + (N_CHUNKS - 1) * CHUNK, CHUNK)], so1
    ).wait()


def kernel(item_id2graph_id, client_id2graph_id, client_ids, item_ids):
    flat_items = item_ids.reshape(-1)
    out_client, out_items = _gather_kernel(
        item_id2graph_id, client_id2graph_id, client_ids, flat_items
    )
    return (out_client, out_items.reshape(BATCH_N, HIST_N))


# speedup vs baseline: 1.2657x; 1.0581x over previous
import functools
import jax
import jax.numpy as jnp
from jax import lax
from jax.experimental import pallas as pl
from jax.experimental.pallas import tpu as pltpu
from jax.experimental.pallas import tpu_sc as plsc

VOCAB_N = 1_000_000
BATCH_N = 16384
HIST_N = 200
TOTAL_ITEMS = BATCH_N * HIST_N

NUM_CORES = 2
NUM_SUBCORES = 16
NW = NUM_CORES * NUM_SUBCORES
ITEMS_PER_W = TOTAL_ITEMS // NW
CLIENTS_PER_W = BATCH_N // NW

CHUNK = 12_800
N_CHUNKS = ITEMS_PER_W // CHUNK

STAGE_PER_SUB = 62_496
STAGE_CHUNK = 15_624
N_STAGE = STAGE_PER_SUB // STAGE_CHUNK
STAGE_TAIL = VOCAB_N - NUM_SUBCORES * STAGE_PER_SUB

_mesh = plsc.VectorSubcoreMesh(core_axis_name="c", subcore_axis_name="s")


@functools.partial(
    pl.kernel,
    out_type=(
        jax.ShapeDtypeStruct((BATCH_N,), jnp.float32),
        jax.ShapeDtypeStruct((TOTAL_ITEMS,), jnp.float32),
    ),
    mesh=_mesh,
    scratch_types=[
        pltpu.VMEM_SHARED((VOCAB_N,), jnp.float32),
        pltpu.VMEM((CHUNK,), jnp.int32),
        pltpu.VMEM((CHUNK,), jnp.int32),
        pltpu.VMEM((CHUNK,), jnp.float32),
        pltpu.VMEM((CHUNK,), jnp.float32),
        pltpu.VMEM((CLIENTS_PER_W,), jnp.int32),
        pltpu.VMEM((CLIENTS_PER_W,), jnp.float32),
        pltpu.VMEM((STAGE_CHUNK,), jnp.float32),
        pltpu.SemaphoreType.DMA,
        pltpu.SemaphoreType.DMA,
        pltpu.SemaphoreType.DMA,
        pltpu.SemaphoreType.DMA,
    ],
)
def _gather_kernel(
    item_tab_hbm,
    client_tab_hbm,
    client_ids_hbm,
    item_ids_hbm,
    out_client_hbm,
    out_items_hbm,
    item_sp,
    idx0_v,
    idx1_v,
    val0_v,
    val1_v,
    cidx_v,
    cval_v,
    stage_v,
    si0,
    si1,
    so0,
    so1,
):
    c = lax.axis_index("c")
    s = lax.axis_index("s")
    wid = s * NUM_CORES + c

    base_off = s * STAGE_PER_SUB

    @pl.loop(0, N_STAGE)
    def _st(j):
        off = base_off + j * STAGE_CHUNK
        pltpu.sync_copy(item_tab_hbm.at[pl.ds(off, STAGE_CHUNK)], stage_v)
        pltpu.sync_copy(stage_v, item_sp.at[pl.ds(off, STAGE_CHUNK)])

    @pl.when(s == 0)
    def _st_tail():
        toff = NUM_SUBCORES * STAGE_PER_SUB
        pltpu.sync_copy(
            item_tab_hbm.at[pl.ds(toff, STAGE_TAIL)],
            stage_v.at[pl.ds(0, STAGE_TAIL)],
        )
        pltpu.sync_copy(
            stage_v.at[pl.ds(0, STAGE_TAIL)],
            item_sp.at[pl.ds(toff, STAGE_TAIL)],
        )

    cbase = wid * CLIENTS_PER_W
    pltpu.sync_copy(client_ids_hbm.at[pl.ds(cbase, CLIENTS_PER_W)], cidx_v)
    pltpu.sync_copy(client_tab_hbm.at[cidx_v], cval_v)
    pltpu.sync_copy(cval_v, out_client_hbm.at[pl.ds(cbase, CLIENTS_PER_W)])

    plsc.subcore_barrier()

    base = wid * ITEMS_PER_W

    pltpu.async_copy(item_ids_hbm.at[pl.ds(base, CHUNK)], idx0_v, si0)
    pltpu.async_copy(item_ids_hbm.at[pl.ds(base + CHUNK, CHUNK)], idx1_v, si1)

    @pl.loop(0, N_CHUNKS, step=2)
    def _chunk(i):
        off0 = base + i * CHUNK
        off1 = off0 + CHUNK

        pltpu.make_async_copy(
            item_ids_hbm.at[pl.ds(off0, CHUNK)], idx0_v, si0
        ).wait()

        @pl.when(i > 0)
        def _wait_out0():
            pltpu.make_async_copy(
                val0_v, out_items_hbm.at[pl.ds(off0 - 2 * CHUNK, CHUNK)], so0
            ).wait()

        pltpu.sync_copy(item_sp.at[idx0_v], val0_v)
        pltpu.async_copy(val0_v, out_items_hbm.at[pl.ds(off0, CHUNK)], so0)

        @pl.when(i + 2 < N_CHUNKS)
        def _prefetch0():
            pltpu.async_copy(
                item_ids_hbm.at[pl.ds(off0 + 2 * CHUNK, CHUNK)], idx0_v, si0
            )

        pltpu.make_async_copy(
            item_ids_hbm.at[pl.ds(off1, CHUNK)], idx1_v, si1
        ).wait()

        @pl.when(i > 0)
        def _wait_out1():
            pltpu.make_async_copy(
                val1_v, out_items_hbm.at[pl.ds(off1 - 2 * CHUNK, CHUNK)], so1
            ).wait()

        pltpu.sync_copy(item_sp.at[idx1_v], val1_v)
        pltpu.async_copy(val1_v, out_items_hbm.at[pl.ds(off1, CHUNK)], so1)

        @pl.when(i + 2 < N_CHUNKS)
        def _prefetch1():
            pltpu.async_copy(
                item_ids_hbm.at[pl.ds(off1 + 2 * CHUNK, CHUNK)], idx1_v, si1
            )

    pltpu.make_async_copy(
        val0_v, out_items_hbm.at[pl.ds(base + (N_CHUNKS - 2) * CHUNK, CHUNK)], so0
    ).wait()
    pltpu.make_async_copy(
        val1_v, out_items_hbm.at[pl.ds(base + (N_CHUNKS - 1) * CHUNK, CHUNK)], so1
    ).wait()


def kernel(item_id2graph_id, client_id2graph_id, client_ids, item_ids):
    flat_items = item_ids.reshape(-1)
    out_client, out_items = _gather_kernel(
        item_id2graph_id, client_id2graph_id, client_ids, flat_items
    )
    return (out_client, out_items.reshape(BATCH_N, HIST_N))
